# baseline (speedup 1.0000x reference)
import functools

import jax
import jax.numpy as jnp
from jax import lax
from jax.experimental import pallas as pl
from jax.experimental.pallas import tpu as pltpu
from jax.experimental.pallas import tpu_sc as plsc

N_NODES = 10000
N_EDGES = 320000
D_IN = 128
H1 = 16
H2 = 32
D_OUT = 40

NC = 2
NS = 16
NW = NC * NS
CHUNK = 128
NCHT = N_EDGES // CHUNK
NCH = 80
NCH_LAST = NCHT - (NW - 1) * NCH
N_PAD = 10112
ROWS_PER_TILE = N_PAD // NS


def _make_spmm(F):
    mesh = plsc.VectorSubcoreMesh(core_axis_name="c", subcore_axis_name="s")

    @functools.partial(
        pl.kernel,
        out_type=jax.ShapeDtypeStruct((NC, N_PAD, F), jnp.float32),
        mesh=mesh,
        scratch_types=[
            pltpu.VMEM((NCH, CHUNK), jnp.int32),
            pltpu.VMEM((NCH, CHUNK), jnp.int32),
            pltpu.VMEM((NCH, CHUNK), jnp.float32),
            pltpu.VMEM((4, CHUNK, F), jnp.float32),
            pltpu.VMEM((CHUNK, F), jnp.float32),
            pltpu.VMEM_SHARED((N_PAD, F), jnp.float32),
            pltpu.VMEM_SHARED((N_PAD, F), jnp.float32),
            [pltpu.SemaphoreType.DMA] * 4,
            pltpu.SemaphoreType.DMA,
            pltpu.SemaphoreType.DMA,
            pltpu.SemaphoreType.DMA,
        ],
        compiler_params=pltpu.CompilerParams(use_tc_tiling_on_sc=False),
    )
    def spmm(feat_hbm, src_hbm, dst_hbm, w_hbm, out_hbm,
             src_v, dst_v, w_v, rows_v, zero_v, acc_sh, feat_sh,
             gsems, esem, fsem, zsem):
        cid = lax.axis_index("c")
        sid = lax.axis_index("s")
        wid = sid * NC + cid

        c0 = wid * NCH

        @pl.when(wid < NW - 1)
        def _():
            pltpu.async_copy(src_hbm.at[pl.ds(c0, NCH)], src_v, esem)
            pltpu.async_copy(dst_hbm.at[pl.ds(c0, NCH)], dst_v, esem)
            pltpu.async_copy(w_hbm.at[pl.ds(c0, NCH)], w_v, esem)

        @pl.when(wid == NW - 1)
        def _():
            pltpu.async_copy(src_hbm.at[pl.ds(c0, NCH_LAST)],
                             src_v.at[pl.ds(0, NCH_LAST)], esem)
            pltpu.async_copy(dst_hbm.at[pl.ds(c0, NCH_LAST)],
                             dst_v.at[pl.ds(0, NCH_LAST)], esem)
            pltpu.async_copy(w_hbm.at[pl.ds(c0, NCH_LAST)],
                             w_v.at[pl.ds(0, NCH_LAST)], esem)
        fbase = sid * (N_NODES // NS)
        pltpu.async_copy(feat_hbm.at[pl.ds(fbase, N_NODES // NS)],
                         feat_sh.at[pl.ds(fbase, N_NODES // NS)], fsem)

        def zrow(i, carry):
            for fb in range(F // 16):
                zero_v[i, pl.ds(fb * 16, 16)] = jnp.zeros((16,), jnp.float32)
            return carry
        lax.fori_loop(0, CHUNK, zrow, 0)
        zbase = sid * ROWS_PER_TILE
        zcopies = []
        off = 0
        while off < ROWS_PER_TILE:
            n = min(CHUNK, ROWS_PER_TILE - off)
            zcopies.append((off, n))
            pltpu.async_copy(zero_v.at[pl.ds(0, n)],
                             acc_sh.at[pl.ds(zbase + off, n)], zsem)
            off += n

        @pl.when(wid < NW - 1)
        def _():
            pltpu.make_async_copy(src_hbm.at[pl.ds(c0, NCH)], src_v,
                                  esem).wait()
            pltpu.make_async_copy(dst_hbm.at[pl.ds(c0, NCH)], dst_v,
                                  esem).wait()
            pltpu.make_async_copy(w_hbm.at[pl.ds(c0, NCH)], w_v,
                                  esem).wait()

        @pl.when(wid == NW - 1)
        def _():
            pltpu.make_async_copy(src_hbm.at[pl.ds(c0, NCH_LAST)],
                                  src_v.at[pl.ds(0, NCH_LAST)], esem).wait()
            pltpu.make_async_copy(dst_hbm.at[pl.ds(c0, NCH_LAST)],
                                  dst_v.at[pl.ds(0, NCH_LAST)], esem).wait()
            pltpu.make_async_copy(w_hbm.at[pl.ds(c0, NCH_LAST)],
                                  w_v.at[pl.ds(0, NCH_LAST)], esem).wait()
        pltpu.make_async_copy(feat_hbm.at[pl.ds(fbase, N_NODES // NS)],
                              feat_sh.at[pl.ds(fbase, N_NODES // NS)],
                              fsem).wait()
        for off, n in zcopies:
            pltpu.make_async_copy(zero_v.at[pl.ds(0, n)],
                                  acc_sh.at[pl.ds(zbase + off, n)],
                                  zsem).wait()
        plsc.subcore_barrier()

        my_nch = jnp.where(wid == NW - 1, NCH_LAST, NCH)

        for b in range(2):
            pltpu.async_copy(feat_sh.at[src_v.at[b]], rows_v.at[b], gsems[b])

        def gather_wait(j, b):
            pltpu.make_async_copy(feat_sh.at[src_v.at[j]], rows_v.at[b],
                                  gsems[b]).wait()

        def round4(i, carry):
            for b in range(4):
                j = 4 * i + b
                buf = rows_v.at[b]
                gather_wait(j, b)

                for g in range(CHUNK // 16):
                    wv = w_v[j, pl.ds(g * 16, 16)]
                    for k in range(16):
                        s = wv[k]
                        e = g * 16 + k
                        for fb in range(F // 16):
                            sl = pl.ds(fb * 16, 16)
                            buf[e, sl] = buf[e, sl] * s

                pltpu.sync_copy(buf, acc_sh.at[dst_v.at[j]], add=True)

                jp = j + 2
                bp = (b + 2) % 4

                @pl.when(jp < my_nch)
                def _():
                    pltpu.async_copy(feat_sh.at[src_v.at[jp]],
                                     rows_v.at[bp], gsems[bp])
            return carry
        lax.fori_loop(0, my_nch // 4, round4, 0)

        plsc.subcore_barrier()
        base = sid * ROWS_PER_TILE
        pltpu.sync_copy(acc_sh.at[pl.ds(base, ROWS_PER_TILE)],
                        out_hbm.at[cid, pl.ds(base, ROWS_PER_TILE)])

    return spmm


_spmm_h1 = _make_spmm(H1)
_spmm_h2 = _make_spmm(H2)


def _tc_in(x, W1):
    def body(x_ref, w_ref, o_ref):
        o_ref[...] = jnp.dot(x_ref[...], w_ref[...],
                             preferred_element_type=jnp.float32)
    return pl.pallas_call(
        body, out_shape=jax.ShapeDtypeStruct((N_NODES, H1), jnp.float32),
    )(x, W1)


def _tc_mid(parts, W2):
    def body(p_ref, w_ref, o_ref):
        h = jnp.maximum(p_ref[0] + p_ref[1], 0.0)
        o_ref[...] = jnp.dot(h, w_ref[...], preferred_element_type=jnp.float32)
    return pl.pallas_call(
        body, out_shape=jax.ShapeDtypeStruct((N_PAD, H2), jnp.float32),
    )(parts, W2)


def _tc_out(parts, W_out, b_out2d):
    def body(p_ref, w_ref, b_ref, o_ref):
        h = jnp.maximum(p_ref[0, :N_NODES] + p_ref[1, :N_NODES], 0.0)
        o_ref[...] = (jnp.dot(h, w_ref[...], preferred_element_type=jnp.float32)
                      + b_ref[...])
    return pl.pallas_call(
        body, out_shape=jax.ShapeDtypeStruct((N_NODES, D_OUT), jnp.float32),
    )(parts, W_out, b_out2d)


def kernel(x, edge_index, edge_weight, W1, W2, W_out, b_out):
    src = edge_index[0].reshape(NCHT, CHUNK)
    dst = edge_index[1].reshape(NCHT, CHUNK)
    w = edge_weight.reshape(NCHT, CHUNK)

    xw = _tc_in(x, W1)
    p1 = _spmm_h1(xw, src, dst, w)
    hw = _tc_mid(p1, W2)
    p2 = _spmm_h2(hw, src, dst, w)
    return _tc_out(p2, W_out, b_out.reshape(1, D_OUT))


# speedup vs baseline: 1.0537x; 1.0537x over previous
import functools

import jax
import jax.numpy as jnp
from jax import lax
from jax.experimental import pallas as pl
from jax.experimental.pallas import tpu as pltpu
from jax.experimental.pallas import tpu_sc as plsc

N_NODES = 10000
N_EDGES = 320000
D_IN = 128
H1 = 16
H2 = 32
D_OUT = 40

NC = 2
NS = 16
NW = NC * NS
CHUNK = 128
NCHT = N_EDGES // CHUNK
NCH = 80
NCH_LAST = NCHT - (NW - 1) * NCH
N_PAD = 10112
ROWS_PER_TILE = N_PAD // NS


def _make_spmm(F):
    mesh = plsc.VectorSubcoreMesh(core_axis_name="c", subcore_axis_name="s")

    @functools.partial(
        pl.kernel,
        out_type=jax.ShapeDtypeStruct((NC, N_PAD, F), jnp.float32),
        mesh=mesh,
        scratch_types=[
            pltpu.VMEM((NCH, CHUNK), jnp.int32),
            pltpu.VMEM((NCH, CHUNK), jnp.int32),
            pltpu.VMEM((NCH, CHUNK), jnp.float32),
            pltpu.VMEM((4, CHUNK, F), jnp.float32),
            pltpu.VMEM((CHUNK, F), jnp.float32),
            pltpu.VMEM_SHARED((N_PAD, F), jnp.float32),
            pltpu.VMEM_SHARED((N_PAD, F), jnp.float32),
            [pltpu.SemaphoreType.DMA] * 4,
            [pltpu.SemaphoreType.DMA] * 4,
            pltpu.SemaphoreType.DMA,
            pltpu.SemaphoreType.DMA,
            pltpu.SemaphoreType.DMA,
        ],
        compiler_params=pltpu.CompilerParams(use_tc_tiling_on_sc=False),
    )
    def spmm(feat_hbm, src_hbm, dst_hbm, w_hbm, out_hbm,
             src_v, dst_v, w_v, rows_v, zero_v, acc_sh, feat_sh,
             gsems, ssems, esem, fsem, zsem):
        cid = lax.axis_index("c")
        sid = lax.axis_index("s")
        wid = sid * NC + cid

        c0 = wid * NCH

        @pl.when(wid < NW - 1)
        def _():
            pltpu.async_copy(src_hbm.at[pl.ds(c0, NCH)], src_v, esem)
            pltpu.async_copy(dst_hbm.at[pl.ds(c0, NCH)], dst_v, esem)
            pltpu.async_copy(w_hbm.at[pl.ds(c0, NCH)], w_v, esem)

        @pl.when(wid == NW - 1)
        def _():
            pltpu.async_copy(src_hbm.at[pl.ds(c0, NCH_LAST)],
                             src_v.at[pl.ds(0, NCH_LAST)], esem)
            pltpu.async_copy(dst_hbm.at[pl.ds(c0, NCH_LAST)],
                             dst_v.at[pl.ds(0, NCH_LAST)], esem)
            pltpu.async_copy(w_hbm.at[pl.ds(c0, NCH_LAST)],
                             w_v.at[pl.ds(0, NCH_LAST)], esem)
        fbase = sid * (N_NODES // NS)
        pltpu.async_copy(feat_hbm.at[pl.ds(fbase, N_NODES // NS)],
                         feat_sh.at[pl.ds(fbase, N_NODES // NS)], fsem)

        def zrow(i, carry):
            for fb in range(F // 16):
                zero_v[i, pl.ds(fb * 16, 16)] = jnp.zeros((16,), jnp.float32)
            return carry
        lax.fori_loop(0, CHUNK, zrow, 0)
        zbase = sid * ROWS_PER_TILE
        zcopies = []
        off = 0
        while off < ROWS_PER_TILE:
            n = min(CHUNK, ROWS_PER_TILE - off)
            zcopies.append((off, n))
            pltpu.async_copy(zero_v.at[pl.ds(0, n)],
                             acc_sh.at[pl.ds(zbase + off, n)], zsem)
            off += n

        @pl.when(wid < NW - 1)
        def _():
            pltpu.make_async_copy(src_hbm.at[pl.ds(c0, NCH)], src_v,
                                  esem).wait()
            pltpu.make_async_copy(dst_hbm.at[pl.ds(c0, NCH)], dst_v,
                                  esem).wait()
            pltpu.make_async_copy(w_hbm.at[pl.ds(c0, NCH)], w_v,
                                  esem).wait()

        @pl.when(wid == NW - 1)
        def _():
            pltpu.make_async_copy(src_hbm.at[pl.ds(c0, NCH_LAST)],
                                  src_v.at[pl.ds(0, NCH_LAST)], esem).wait()
            pltpu.make_async_copy(dst_hbm.at[pl.ds(c0, NCH_LAST)],
                                  dst_v.at[pl.ds(0, NCH_LAST)], esem).wait()
            pltpu.make_async_copy(w_hbm.at[pl.ds(c0, NCH_LAST)],
                                  w_v.at[pl.ds(0, NCH_LAST)], esem).wait()
        pltpu.make_async_copy(feat_hbm.at[pl.ds(fbase, N_NODES // NS)],
                              feat_sh.at[pl.ds(fbase, N_NODES // NS)],
                              fsem).wait()
        for off, n in zcopies:
            pltpu.make_async_copy(zero_v.at[pl.ds(0, n)],
                                  acc_sh.at[pl.ds(zbase + off, n)],
                                  zsem).wait()
        plsc.subcore_barrier()

        my_nch = jnp.where(wid == NW - 1, NCH_LAST, NCH)

        for b in range(2):
            pltpu.async_copy(feat_sh.at[src_v.at[b]], rows_v.at[b], gsems[b])

        def gather_wait(j, b):
            pltpu.make_async_copy(feat_sh.at[src_v.at[j]], rows_v.at[b],
                                  gsems[b]).wait()

        def process(j, b, wait_scatter):
            buf = rows_v.at[b]
            gather_wait(j, b)

            for g in range(CHUNK // 16):
                wv = w_v[j, pl.ds(g * 16, 16)]
                for k in range(16):
                    s = wv[k]
                    e = g * 16 + k
                    for fb in range(F // 16):
                        sl = pl.ds(fb * 16, 16)
                        buf[e, sl] = buf[e, sl] * s

            pltpu.async_copy(buf, acc_sh.at[dst_v.at[j]], ssems[b],
                             add=True)

            jp = j + 2
            bp = (b + 2) % 4

            @pl.when(jp < my_nch)
            def _():
                if wait_scatter:
                    pltpu.make_async_copy(rows_v.at[bp],
                                          acc_sh.at[dst_v.at[j - 2]],
                                          ssems[bp]).wait()
                pltpu.async_copy(feat_sh.at[src_v.at[jp]],
                                 rows_v.at[bp], gsems[bp])

        for b in range(4):
            process(b, b, b >= 2)

        def round4(i, carry):
            for b in range(4):
                process(4 * i + b, b, True)
            return carry
        lax.fori_loop(1, my_nch // 4, round4, 0)

        for b in range(4):
            pltpu.make_async_copy(rows_v.at[b],
                                  acc_sh.at[dst_v.at[my_nch - 4 + b]],
                                  ssems[b]).wait()

        plsc.subcore_barrier()
        base = sid * ROWS_PER_TILE
        pltpu.sync_copy(acc_sh.at[pl.ds(base, ROWS_PER_TILE)],
                        out_hbm.at[cid, pl.ds(base, ROWS_PER_TILE)])

    return spmm


_spmm_h1 = _make_spmm(H1)
_spmm_h2 = _make_spmm(H2)


def _tc_in(x, W1):
    def body(x_ref, w_ref, o_ref):
        o_ref[...] = jnp.dot(x_ref[...], w_ref[...],
                             preferred_element_type=jnp.float32)
    return pl.pallas_call(
        body, out_shape=jax.ShapeDtypeStruct((N_NODES, H1), jnp.float32),
    )(x, W1)


def _tc_mid(parts, W2):
    def body(p_ref, w_ref, o_ref):
        h = jnp.maximum(p_ref[0] + p_ref[1], 0.0)
        o_ref[...] = jnp.dot(h, w_ref[...], preferred_element_type=jnp.float32)
    return pl.pallas_call(
        body, out_shape=jax.ShapeDtypeStruct((N_PAD, H2), jnp.float32),
    )(parts, W2)


def _tc_out(parts, W_out, b_out2d):
    def body(p_ref, w_ref, b_ref, o_ref):
        h = jnp.maximum(p_ref[0, :N_NODES] + p_ref[1, :N_NODES], 0.0)
        o_ref[...] = (jnp.dot(h, w_ref[...], preferred_element_type=jnp.float32)
                      + b_ref[...])
    return pl.pallas_call(
        body, out_shape=jax.ShapeDtypeStruct((N_NODES, D_OUT), jnp.float32),
    )(parts, W_out, b_out2d)


def kernel(x, edge_index, edge_weight, W1, W2, W_out, b_out):
    src = edge_index[0].reshape(NCHT, CHUNK)
    dst = edge_index[1].reshape(NCHT, CHUNK)
    w = edge_weight.reshape(NCHT, CHUNK)

    xw = _tc_in(x, W1)
    p1 = _spmm_h1(xw, src, dst, w)
    hw = _tc_mid(p1, W2)
    p2 = _spmm_h2(hw, src, dst, w)
    return _tc_out(p2, W_out, b_out.reshape(1, D_OUT))
